# baseline (speedup 1.0000x reference)
import functools

import jax
import jax.numpy as jnp
from jax import lax
from jax.experimental import pallas as pl
from jax.experimental.pallas import tpu as pltpu
from jax.experimental.pallas import tpu_sc as plsc

N_NODES = 10000
N_EDGES = 320000
N_PAIRS = 100000
D = 128

NC = 2
NS = 16
NW = NC * NS
NP = 10240
STRIPE = NP // NS

EB = 128
EP = 327680
EBA = 128
AGG_W = NS
ROWS_PER_W = EP // EBA // AGG_W
CH = 16
NCH = ROWS_PER_W // CH
ROWS_PER_T = EP // EB // NS

PP = 102400
P_PER_W = PP // NW
P_ITER = P_PER_W // EB
DP = 16

BLK = 1024
GRID = NP // BLK

_mesh = plsc.VectorSubcoreMesh(core_axis_name="c", subcore_axis_name="s")


def _deg_body(eidx_h, ones_h, zvec_h, out_h, idx_v, ones_v, deg_sh):
    cid = lax.axis_index("c")
    sid = lax.axis_index("s")
    pltpu.sync_copy(ones_h, ones_v)
    pltpu.sync_copy(zvec_h, deg_sh.at[pl.ds(sid * STRIPE, STRIPE)])
    for m in range(2):
        pltpu.sync_copy(eidx_h.at[cid, 2 * sid + m],
                        idx_v.at[pl.ds(m * (ROWS_PER_T // 2), ROWS_PER_T // 2)])
    plsc.subcore_barrier()

    def step(j, _):
        pltpu.sync_copy(ones_v, deg_sh.at[idx_v.at[j]], add=True)
        return 0

    lax.fori_loop(0, ROWS_PER_T, step, 0)
    plsc.subcore_barrier()
    pltpu.sync_copy(deg_sh.at[pl.ds(sid * STRIPE, STRIPE)],
                    out_h.at[cid, pl.ds(sid * STRIPE, STRIPE)])


def _agg_body(tab_h, srcb_h, dstb_h, zblk_h, out_h,
              idx_s, idx_d, rows0, rows1, acc_sh, sem0, sem1):
    sid = lax.axis_index("s")
    wid = sid
    pltpu.sync_copy(zblk_h, acc_sh.at[pl.ds(sid * STRIPE, STRIPE)])
    plsc.subcore_barrier()

    def chunk(c, _):
        pltpu.sync_copy(srcb_h.at[wid, pl.ds(c * CH, CH)], idx_s)
        pltpu.sync_copy(dstb_h.at[wid, pl.ds(c * CH, CH)], idx_d)
        pltpu.async_copy(tab_h.at[idx_s.at[0]], rows0, sem0)

        def step(jj, _):
            j = 2 * jj
            pltpu.async_copy(tab_h.at[idx_s.at[j + 1]], rows1, sem1)
            pltpu.make_async_copy(tab_h.at[idx_s.at[j]], rows0, sem0).wait()
            pltpu.sync_copy(rows0, acc_sh.at[idx_d.at[j]], add=True)

            @pl.when(jj < CH // 2 - 1)
            def _():
                pltpu.async_copy(tab_h.at[idx_s.at[j + 2]], rows0, sem0)

            pltpu.make_async_copy(tab_h.at[idx_s.at[j + 1]], rows1, sem1).wait()
            pltpu.sync_copy(rows1, acc_sh.at[idx_d.at[j + 1]], add=True)
            return 0

        lax.fori_loop(0, CH // 2, step, 0)
        return 0

    lax.fori_loop(0, NCH, chunk, 0)
    plsc.subcore_barrier()
    pltpu.sync_copy(acc_sh.at[pl.ds(sid * STRIPE, STRIPE)],
                    out_h.at[pl.ds(sid * STRIPE, STRIPE)])


def _pair_body(u_h, v_h, p0_h, p1_h, o_h,
               p0_v, p1_v, ur_v, vr_v, o_v, semu, semv):
    cid = lax.axis_index("c")
    sid = lax.axis_index("s")
    wid = cid * NS + sid
    pltpu.sync_copy(p0_h.at[wid], p0_v)
    pltpu.sync_copy(p1_h.at[wid], p1_v)

    def step(j, _):
        cu = pltpu.async_copy(u_h.at[p0_v.at[j]], ur_v, semu)
        cv = pltpu.async_copy(v_h.at[p1_v.at[j]], vr_v, semv)
        cu.wait()
        cv.wait()

        def add_row(k, _):
            o_v[k, :] = ur_v[k, :] + vr_v[k, :]
            return 0

        lax.fori_loop(0, EB, add_row, 0)
        pltpu.sync_copy(o_v, o_h.at[wid, pl.ds(j * EB, EB)])
        return 0

    lax.fori_loop(0, P_ITER, step, 0)


_deg_call = pl.kernel(
    _deg_body,
    out_type=jax.ShapeDtypeStruct((NC, NP), jnp.float32),
    mesh=_mesh,
    scratch_types=[
        pltpu.VMEM((ROWS_PER_T, EB), jnp.int32),
        pltpu.VMEM((EB,), jnp.float32),
        pltpu.VMEM_SHARED((NP,), jnp.float32),
    ],
)

_agg_mesh = plsc.VectorSubcoreMesh(core_axis_name="c", subcore_axis_name="s",
                                   num_cores=1)

_agg_call = pl.kernel(
    _agg_body,
    out_type=jax.ShapeDtypeStruct((NP, D), jnp.float32),
    mesh=_agg_mesh,
    scratch_types=[
        pltpu.VMEM((CH, EBA), jnp.int32),
        pltpu.VMEM((CH, EBA), jnp.int32),
        pltpu.VMEM((EBA, D), jnp.float32),
        pltpu.VMEM((EBA, D), jnp.float32),
        pltpu.VMEM_SHARED((NP, D), jnp.float32),
        pltpu.SemaphoreType.DMA,
        pltpu.SemaphoreType.DMA,
    ],
)

_pair_call = pl.kernel(
    _pair_body,
    out_type=jax.ShapeDtypeStruct((NW, P_PER_W, DP), jnp.float32),
    mesh=_mesh,
    scratch_types=[
        pltpu.VMEM((P_ITER, EB), jnp.int32),
        pltpu.VMEM((P_ITER, EB), jnp.int32),
        pltpu.VMEM((EB, DP), jnp.float32),
        pltpu.VMEM((EB, DP), jnp.float32),
        pltpu.VMEM((EB, DP), jnp.float32),
        pltpu.SemaphoreType.DMA,
        pltpu.SemaphoreType.DMA,
    ],
    compiler_params=pltpu.CompilerParams(use_tc_tiling_on_sc=False),
)


def _scale_body(x_ref, dT_ref, o_ref):
    a = lax.rsqrt(jnp.maximum(dT_ref[:, 0:1], 1.0))
    o_ref[...] = x_ref[...] * a


def _mm1_body(p_ref, dT_ref, W_ref, b_ref, o_ref):
    acc = p_ref[...]
    dT = dT_ref[...]
    bsc = lax.rsqrt(jnp.maximum(dT[:, 1:2], 1.0))
    asc = lax.rsqrt(jnp.maximum(dT[:, 0:1], 1.0))
    y = jnp.dot(acc * bsc, W_ref[...], preferred_element_type=jnp.float32)
    o_ref[...] = jnp.maximum(y + b_ref[...], 0.0) * asc


def _mm2_body(p_ref, dT_ref, W_ref, b_ref, wt_ref, wb_ref, bc_ref,
              z_ref, u_ref, v_ref):
    acc = p_ref[...]
    dT = dT_ref[...]
    bsc = lax.rsqrt(jnp.maximum(dT[:, 1:2], 1.0))
    y = jnp.dot(acc * bsc, W_ref[...], preferred_element_type=jnp.float32)
    z = jnp.maximum(y + b_ref[...], 0.0)
    z_ref[...] = z
    u_ref[...] = jnp.dot(z, wt_ref[...], preferred_element_type=jnp.float32) + bc_ref[...]
    v_ref[...] = jnp.dot(z, wb_ref[...], preferred_element_type=jnp.float32)


def _row_spec():
    return pl.BlockSpec((BLK, D), lambda i: (i, 0))


def _deg_spec():
    return pl.BlockSpec((BLK, 2), lambda i: (i, 0))


def _full(shape):
    return pl.BlockSpec(shape, lambda i: tuple(0 for _ in shape))


_scale_call = pl.pallas_call(
    _scale_body,
    grid=(GRID,),
    in_specs=[_row_spec(), _deg_spec()],
    out_specs=_row_spec(),
    out_shape=jax.ShapeDtypeStruct((NP, D), jnp.float32),
)

_mm1_call = pl.pallas_call(
    _mm1_body,
    grid=(GRID,),
    in_specs=[_row_spec(), _deg_spec(),
              _full((D, D)), _full((1, D))],
    out_specs=_row_spec(),
    out_shape=jax.ShapeDtypeStruct((NP, D), jnp.float32),
)

_mm2_call = pl.pallas_call(
    _mm2_body,
    grid=(GRID,),
    in_specs=[_row_spec(), _deg_spec(),
              _full((D, D)), _full((1, D)),
              _full((D, DP)), _full((D, DP)), _full((1, DP))],
    out_specs=[_row_spec(),
               pl.BlockSpec((BLK, DP), lambda i: (i, 0)),
               pl.BlockSpec((BLK, DP), lambda i: (i, 0))],
    out_shape=[jax.ShapeDtypeStruct((NP, D), jnp.float32),
               jax.ShapeDtypeStruct((NP, DP), jnp.float32),
               jax.ShapeDtypeStruct((NP, DP), jnp.float32)],
)


def kernel(x, edge_index, pair_index, W1, b1, W2, b2, Wc, bc):
    f32 = jnp.float32
    x_pad = jnp.pad(x, ((0, NP - N_NODES), (0, 0)))

    pad_e = jnp.full((2, EP - N_EDGES), N_NODES, jnp.int32)
    e = jnp.concatenate([edge_index, pad_e], axis=1)
    eb = e.reshape(2, AGG_W, ROWS_PER_W, EBA)
    ebt = e.reshape(2, NS * 2, ROWS_PER_T // 2, EB)

    pad_p = jnp.zeros((2, PP - N_PAIRS), jnp.int32)
    p = jnp.concatenate([pair_index, pad_p], axis=1).reshape(2, NW, P_ITER, EB)

    ones_e = jnp.ones((EB,), f32)
    zvec = jnp.zeros((STRIPE,), f32)
    zblk = jnp.zeros((STRIPE, D), f32)

    deg = _deg_call(ebt, ones_e, zvec)
    degT = deg.T

    hs1 = _scale_call(x_pad, degT)
    p1 = _agg_call(hs1, eb[0], eb[1], zblk)
    h1 = _mm1_call(p1, degT, W1, b1.reshape(1, D))
    p2 = _agg_call(h1, eb[0], eb[1], zblk)
    wt = jnp.pad(Wc[:D], ((0, 0), (0, DP - 2)))
    wb = jnp.pad(Wc[D:], ((0, 0), (0, DP - 2)))
    bcp = jnp.pad(bc.reshape(1, 2), ((0, 0), (0, DP - 2)))
    z, u, v = _mm2_call(p2, degT, W2, b2.reshape(1, D), wt, wb, bcp)

    o = _pair_call(u, v, p[0], p[1])
    logits = o.reshape(PP, DP)[:N_PAIRS, :2]
    return (z[:N_NODES], logits)


# speedup vs baseline: 1.0667x; 1.0667x over previous
import functools

import jax
import jax.numpy as jnp
from jax import lax
from jax.experimental import pallas as pl
from jax.experimental.pallas import tpu as pltpu
from jax.experimental.pallas import tpu_sc as plsc

N_NODES = 10000
N_EDGES = 320000
N_PAIRS = 100000
D = 128

NC = 2
NS = 16
NW = NC * NS
NP = 10240
STRIPE = NP // NS

EB = 128
EP = 327680
EBA = 128
AGG_W = NW
ROWS_PER_W = EP // EBA // AGG_W
ROWS_PER_T = EP // EB // NS

PP = 102400
P_PER_W = PP // NW
P_ITER = P_PER_W // EB
DP = 16

BLK = 1024
GRID = NP // BLK

_mesh = plsc.VectorSubcoreMesh(core_axis_name="c", subcore_axis_name="s")


def _deg_body(eidx_h, ones_h, zvec_h, out_h, idx_v, ones_v, deg_sh):
    cid = lax.axis_index("c")
    sid = lax.axis_index("s")
    pltpu.sync_copy(ones_h, ones_v)
    pltpu.sync_copy(zvec_h, deg_sh.at[pl.ds(sid * STRIPE, STRIPE)])
    for m in range(2):
        pltpu.sync_copy(eidx_h.at[cid, 2 * sid + m],
                        idx_v.at[pl.ds(m * (ROWS_PER_T // 2), ROWS_PER_T // 2)])
    plsc.subcore_barrier()

    def step(j, _):
        pltpu.sync_copy(ones_v, deg_sh.at[idx_v.at[j]], add=True)
        return 0

    lax.fori_loop(0, ROWS_PER_T, step, 0)
    plsc.subcore_barrier()
    pltpu.sync_copy(deg_sh.at[pl.ds(sid * STRIPE, STRIPE)],
                    out_h.at[cid, pl.ds(sid * STRIPE, STRIPE)])


def _agg_body(tab_h, srcb_h, dstb_h, zblk_h, out_h,
              idx_s, idx_d, rows_v, acc_sh, sem):
    cid = lax.axis_index("c")
    sid = lax.axis_index("s")
    wid = cid * NS + sid
    pltpu.sync_copy(zblk_h, acc_sh.at[pl.ds(sid * STRIPE, STRIPE)])
    pltpu.sync_copy(srcb_h.at[wid], idx_s)
    pltpu.sync_copy(dstb_h.at[wid], idx_d)
    plsc.subcore_barrier()

    def step(j, _):
        pltpu.async_copy(tab_h.at[idx_s.at[j]], rows_v, sem).wait()
        pltpu.sync_copy(rows_v, acc_sh.at[idx_d.at[j]], add=True)
        return 0

    lax.fori_loop(0, ROWS_PER_W, step, 0)
    plsc.subcore_barrier()
    pltpu.sync_copy(acc_sh.at[pl.ds(sid * STRIPE, STRIPE)],
                    out_h.at[cid, pl.ds(sid * STRIPE, STRIPE)])


def _pair_body(u_h, v_h, p0_h, p1_h, o_h,
               p0_v, p1_v, ur_v, vr_v, o_v, semu, semv):
    cid = lax.axis_index("c")
    sid = lax.axis_index("s")
    wid = cid * NS + sid
    pltpu.sync_copy(p0_h.at[wid], p0_v)
    pltpu.sync_copy(p1_h.at[wid], p1_v)

    def step(j, _):
        cu = pltpu.async_copy(u_h.at[p0_v.at[j]], ur_v, semu)
        cv = pltpu.async_copy(v_h.at[p1_v.at[j]], vr_v, semv)
        cu.wait()
        cv.wait()

        def add_row(k, _):
            o_v[k, :] = ur_v[k, :] + vr_v[k, :]
            return 0

        lax.fori_loop(0, EB, add_row, 0)
        pltpu.sync_copy(o_v, o_h.at[wid, pl.ds(j * EB, EB)])
        return 0

    lax.fori_loop(0, P_ITER, step, 0)


_deg_call = pl.kernel(
    _deg_body,
    out_type=jax.ShapeDtypeStruct((NC, NP), jnp.float32),
    mesh=_mesh,
    scratch_types=[
        pltpu.VMEM((ROWS_PER_T, EB), jnp.int32),
        pltpu.VMEM((EB,), jnp.float32),
        pltpu.VMEM_SHARED((NP,), jnp.float32),
    ],
)

_agg_call = pl.kernel(
    _agg_body,
    out_type=jax.ShapeDtypeStruct((NC, NP, D), jnp.float32),
    mesh=_mesh,
    scratch_types=[
        pltpu.VMEM((ROWS_PER_W, EBA), jnp.int32),
        pltpu.VMEM((ROWS_PER_W, EBA), jnp.int32),
        pltpu.VMEM((EBA, D), jnp.float32),
        pltpu.VMEM_SHARED((NP, D), jnp.float32),
        pltpu.SemaphoreType.DMA,
    ],
)

_pair_call = pl.kernel(
    _pair_body,
    out_type=jax.ShapeDtypeStruct((NW, P_PER_W, DP), jnp.float32),
    mesh=_mesh,
    scratch_types=[
        pltpu.VMEM((P_ITER, EB), jnp.int32),
        pltpu.VMEM((P_ITER, EB), jnp.int32),
        pltpu.VMEM((EB, DP), jnp.float32),
        pltpu.VMEM((EB, DP), jnp.float32),
        pltpu.VMEM((EB, DP), jnp.float32),
        pltpu.SemaphoreType.DMA,
        pltpu.SemaphoreType.DMA,
    ],
    compiler_params=pltpu.CompilerParams(use_tc_tiling_on_sc=False),
)


def _scale_body(x_ref, dT_ref, o_ref):
    a = lax.rsqrt(jnp.maximum(dT_ref[:, 0:1], 1.0))
    o_ref[...] = x_ref[...] * a


def _mm1_body(p_ref, dT_ref, W_ref, b_ref, o_ref):
    acc = p_ref[0] + p_ref[1]
    dT = dT_ref[...]
    bsc = lax.rsqrt(jnp.maximum(dT[:, 1:2], 1.0))
    asc = lax.rsqrt(jnp.maximum(dT[:, 0:1], 1.0))
    y = jnp.dot(acc * bsc, W_ref[...], preferred_element_type=jnp.float32)
    o_ref[...] = jnp.maximum(y + b_ref[...], 0.0) * asc


def _mm2_body(p_ref, dT_ref, W_ref, b_ref, wt_ref, wb_ref, bc_ref,
              z_ref, u_ref, v_ref):
    acc = p_ref[0] + p_ref[1]
    dT = dT_ref[...]
    bsc = lax.rsqrt(jnp.maximum(dT[:, 1:2], 1.0))
    y = jnp.dot(acc * bsc, W_ref[...], preferred_element_type=jnp.float32)
    z = jnp.maximum(y + b_ref[...], 0.0)
    z_ref[...] = z
    u_ref[...] = jnp.dot(z, wt_ref[...], preferred_element_type=jnp.float32) + bc_ref[...]
    v_ref[...] = jnp.dot(z, wb_ref[...], preferred_element_type=jnp.float32)


def _row_spec():
    return pl.BlockSpec((BLK, D), lambda i: (i, 0))


def _deg_spec():
    return pl.BlockSpec((BLK, 2), lambda i: (i, 0))


def _full(shape):
    return pl.BlockSpec(shape, lambda i: tuple(0 for _ in shape))


_scale_call = pl.pallas_call(
    _scale_body,
    grid=(GRID,),
    in_specs=[_row_spec(), _deg_spec()],
    out_specs=_row_spec(),
    out_shape=jax.ShapeDtypeStruct((NP, D), jnp.float32),
)

_mm1_call = pl.pallas_call(
    _mm1_body,
    grid=(GRID,),
    in_specs=[pl.BlockSpec((2, BLK, D), lambda i: (0, i, 0)), _deg_spec(),
              _full((D, D)), _full((1, D))],
    out_specs=_row_spec(),
    out_shape=jax.ShapeDtypeStruct((NP, D), jnp.float32),
)

_mm2_call = pl.pallas_call(
    _mm2_body,
    grid=(GRID,),
    in_specs=[pl.BlockSpec((2, BLK, D), lambda i: (0, i, 0)), _deg_spec(),
              _full((D, D)), _full((1, D)),
              _full((D, DP)), _full((D, DP)), _full((1, DP))],
    out_specs=[_row_spec(),
               pl.BlockSpec((BLK, DP), lambda i: (i, 0)),
               pl.BlockSpec((BLK, DP), lambda i: (i, 0))],
    out_shape=[jax.ShapeDtypeStruct((NP, D), jnp.float32),
               jax.ShapeDtypeStruct((NP, DP), jnp.float32),
               jax.ShapeDtypeStruct((NP, DP), jnp.float32)],
)


def kernel(x, edge_index, pair_index, W1, b1, W2, b2, Wc, bc):
    f32 = jnp.float32
    x_pad = jnp.pad(x, ((0, NP - N_NODES), (0, 0)))

    pad_e = jnp.full((2, EP - N_EDGES), N_NODES, jnp.int32)
    e = jnp.concatenate([edge_index, pad_e], axis=1)
    eb = e.reshape(2, AGG_W, ROWS_PER_W, EBA)
    ebt = e.reshape(2, NS * 2, ROWS_PER_T // 2, EB)

    pad_p = jnp.zeros((2, PP - N_PAIRS), jnp.int32)
    p = jnp.concatenate([pair_index, pad_p], axis=1).reshape(2, NW, P_ITER, EB)

    ones_e = jnp.ones((EB,), f32)
    zvec = jnp.zeros((STRIPE,), f32)
    zblk = jnp.zeros((STRIPE, D), f32)

    deg = _deg_call(ebt, ones_e, zvec)
    degT = deg.T

    hs1 = _scale_call(x_pad, degT)
    p1 = _agg_call(hs1, eb[0], eb[1], zblk)
    h1 = _mm1_call(p1, degT, W1, b1.reshape(1, D))
    p2 = _agg_call(h1, eb[0], eb[1], zblk)
    wt = jnp.pad(Wc[:D], ((0, 0), (0, DP - 2)))
    wb = jnp.pad(Wc[D:], ((0, 0), (0, DP - 2)))
    bcp = jnp.pad(bc.reshape(1, 2), ((0, 0), (0, DP - 2)))
    z, u, v = _mm2_call(p2, degT, W2, b2.reshape(1, D), wt, wb, bcp)

    o = _pair_call(u, v, p[0], p[1])
    logits = o.reshape(PP, DP)[:N_PAIRS, :2]
    return (z[:N_NODES], logits)


# speedup vs baseline: 2.3540x; 2.2067x over previous
import functools

import jax
import jax.numpy as jnp
from jax import lax
from jax.experimental import pallas as pl
from jax.experimental.pallas import tpu as pltpu
from jax.experimental.pallas import tpu_sc as plsc

N_NODES = 10000
N_EDGES = 320000
N_PAIRS = 100000
D = 128

NC = 2
NS = 16
NW = NC * NS
NP = 10240
STRIPE = NP // NS

EB = 128
EP = 327680
EBA = 128
AGG_W = NW
ROWS_PER_W = EP // EBA // AGG_W
ROWS_PER_T = EP // EB // NS

PP = 102400
P_PER_W = PP // NW
P_ITER = P_PER_W // EB
DP = 16

BLK = 1024
GRID = NP // BLK

_mesh = plsc.VectorSubcoreMesh(core_axis_name="c", subcore_axis_name="s")


def _deg_body(eidx_h, ones_h, zvec_h, out_h, idx_v, ones_v, deg_sh):
    cid = lax.axis_index("c")
    sid = lax.axis_index("s")
    pltpu.sync_copy(ones_h, ones_v)
    pltpu.sync_copy(zvec_h, deg_sh.at[pl.ds(sid * STRIPE, STRIPE)])
    for m in range(2):
        pltpu.sync_copy(eidx_h.at[cid, 2 * sid + m],
                        idx_v.at[pl.ds(m * (ROWS_PER_T // 2), ROWS_PER_T // 2)])
    plsc.subcore_barrier()

    def step(j, _):
        pltpu.sync_copy(ones_v, deg_sh.at[idx_v.at[j]], add=True)
        return 0

    lax.fori_loop(0, ROWS_PER_T, step, 0)
    plsc.subcore_barrier()
    pltpu.sync_copy(deg_sh.at[pl.ds(sid * STRIPE, STRIPE)],
                    out_h.at[cid, pl.ds(sid * STRIPE, STRIPE)])


def _agg_body(tab_h, srcb_h, dstb_h, zblk_h, out_h,
              idx_s, idx_d, rows_v, acc_sh, sem):
    cid = lax.axis_index("c")
    sid = lax.axis_index("s")
    wid = cid * NS + sid
    pltpu.sync_copy(zblk_h, acc_sh.at[pl.ds(sid * STRIPE, STRIPE)])
    pltpu.sync_copy(srcb_h.at[wid], idx_s)
    pltpu.sync_copy(dstb_h.at[wid], idx_d)
    plsc.subcore_barrier()

    def step(j, _):
        pltpu.async_copy(tab_h.at[idx_s.at[j]], rows_v, sem).wait()
        pltpu.sync_copy(rows_v, acc_sh.at[idx_d.at[j]], add=True)
        return 0

    lax.fori_loop(0, ROWS_PER_W, step, 0)
    plsc.subcore_barrier()
    pltpu.sync_copy(acc_sh.at[pl.ds(sid * STRIPE, STRIPE)],
                    out_h.at[cid, pl.ds(sid * STRIPE, STRIPE)])


def _pair_body(u_h, v_h, p0_h, p1_h, o_h,
               p0_v, p1_v, ur_v, vr_v, o_v, semu, semv):
    cid = lax.axis_index("c")
    sid = lax.axis_index("s")
    wid = cid * NS + sid
    pltpu.sync_copy(p0_h.at[wid], p0_v)
    pltpu.sync_copy(p1_h.at[wid], p1_v)

    def step(j, _):
        cu = pltpu.async_copy(u_h.at[p0_v.at[j]], ur_v, semu)
        cv = pltpu.async_copy(v_h.at[p1_v.at[j]], vr_v, semv)
        cu.wait()
        cv.wait()

        def add_row(k, _):
            o_v[k, :] = ur_v[k, :] + vr_v[k, :]
            return 0

        lax.fori_loop(0, EB, add_row, 0)
        pltpu.sync_copy(o_v, o_h.at[wid, pl.ds(j * EB, EB)])
        return 0

    lax.fori_loop(0, P_ITER, step, 0)


_deg_call = pl.kernel(
    _deg_body,
    out_type=jax.ShapeDtypeStruct((NC, NP), jnp.float32),
    mesh=_mesh,
    scratch_types=[
        pltpu.VMEM((ROWS_PER_T, EB), jnp.int32),
        pltpu.VMEM((EB,), jnp.float32),
        pltpu.VMEM_SHARED((NP,), jnp.float32),
    ],
)

_agg_call = pl.kernel(
    _agg_body,
    out_type=jax.ShapeDtypeStruct((NC, NP, D), jnp.float32),
    mesh=_mesh,
    scratch_types=[
        pltpu.VMEM((ROWS_PER_W, EBA), jnp.int32),
        pltpu.VMEM((ROWS_PER_W, EBA), jnp.int32),
        pltpu.VMEM((EBA, D), jnp.float32),
        pltpu.VMEM_SHARED((NP, D), jnp.float32),
        pltpu.SemaphoreType.DMA,
    ],
)

_pair_call = pl.kernel(
    _pair_body,
    out_type=jax.ShapeDtypeStruct((NW, P_PER_W, DP), jnp.float32),
    mesh=_mesh,
    scratch_types=[
        pltpu.VMEM((P_ITER, EB), jnp.int32),
        pltpu.VMEM((P_ITER, EB), jnp.int32),
        pltpu.VMEM((EB, DP), jnp.float32),
        pltpu.VMEM((EB, DP), jnp.float32),
        pltpu.VMEM((EB, DP), jnp.float32),
        pltpu.SemaphoreType.DMA,
        pltpu.SemaphoreType.DMA,
    ],
    compiler_params=pltpu.CompilerParams(use_tc_tiling_on_sc=False),
)


def _scale_body(x_ref, dT_ref, o_ref):
    a = lax.rsqrt(jnp.maximum(dT_ref[:, 0:1], 1.0))
    o_ref[...] = x_ref[...] * a


def _mm1_body(p_ref, dT_ref, W_ref, b_ref, o_ref):
    acc = p_ref[0] + p_ref[1]
    dT = dT_ref[...]
    bsc = lax.rsqrt(jnp.maximum(dT[:, 1:2], 1.0))
    asc = lax.rsqrt(jnp.maximum(dT[:, 0:1], 1.0))
    y = jnp.dot(acc * bsc, W_ref[...], preferred_element_type=jnp.float32)
    o_ref[...] = jnp.maximum(y + b_ref[...], 0.0) * asc


def _mm2_body(p_ref, dT_ref, W_ref, b_ref, wt_ref, wb_ref, bc_ref,
              z_ref, u_ref, v_ref):
    acc = p_ref[0] + p_ref[1]
    dT = dT_ref[...]
    bsc = lax.rsqrt(jnp.maximum(dT[:, 1:2], 1.0))
    y = jnp.dot(acc * bsc, W_ref[...], preferred_element_type=jnp.float32)
    z = jnp.maximum(y + b_ref[...], 0.0)
    z_ref[...] = z
    u_ref[...] = jnp.dot(z, wt_ref[...], preferred_element_type=jnp.float32) + bc_ref[...]
    v_ref[...] = jnp.dot(z, wb_ref[...], preferred_element_type=jnp.float32)


def _row_spec():
    return pl.BlockSpec((BLK, D), lambda i: (i, 0))


def _deg_spec():
    return pl.BlockSpec((BLK, 2), lambda i: (i, 0))


def _full(shape):
    return pl.BlockSpec(shape, lambda i: tuple(0 for _ in shape))


_scale_call = pl.pallas_call(
    _scale_body,
    grid=(GRID,),
    in_specs=[_row_spec(), _deg_spec()],
    out_specs=_row_spec(),
    out_shape=jax.ShapeDtypeStruct((NP, D), jnp.float32),
)

_mm1_call = pl.pallas_call(
    _mm1_body,
    grid=(GRID,),
    in_specs=[pl.BlockSpec((2, BLK, D), lambda i: (0, i, 0)), _deg_spec(),
              _full((D, D)), _full((1, D))],
    out_specs=_row_spec(),
    out_shape=jax.ShapeDtypeStruct((NP, D), jnp.float32),
)

_mm2_call = pl.pallas_call(
    _mm2_body,
    grid=(GRID,),
    in_specs=[pl.BlockSpec((2, BLK, D), lambda i: (0, i, 0)), _deg_spec(),
              _full((D, D)), _full((1, D)),
              _full((D, DP)), _full((D, DP)), _full((1, DP))],
    out_specs=[_row_spec(),
               pl.BlockSpec((BLK, DP), lambda i: (i, 0)),
               pl.BlockSpec((BLK, DP), lambda i: (i, 0))],
    out_shape=[jax.ShapeDtypeStruct((NP, D), jnp.float32),
               jax.ShapeDtypeStruct((NP, DP), jnp.float32),
               jax.ShapeDtypeStruct((NP, DP), jnp.float32)],
)


def kernel(x, edge_index, pair_index, W1, b1, W2, b2, Wc, bc):
    f32 = jnp.float32
    x_pad = jnp.pad(x, ((0, NP - N_NODES), (0, 0)))

    pad_ids = N_NODES + jnp.arange(EP - N_EDGES, dtype=jnp.int32) % (NP - N_NODES)
    e = jnp.concatenate([edge_index, jnp.stack([pad_ids, pad_ids])], axis=1)
    eb = e.reshape(2, AGG_W, ROWS_PER_W, EBA)
    ebt = e.reshape(2, NS * 2, ROWS_PER_T // 2, EB)

    pad_pids = jnp.arange(PP - N_PAIRS, dtype=jnp.int32) % N_NODES
    p = jnp.concatenate([pair_index, jnp.stack([pad_pids, pad_pids])],
                        axis=1).reshape(2, NW, P_ITER, EB)

    ones_e = jnp.ones((EB,), f32)
    zvec = jnp.zeros((STRIPE,), f32)
    zblk = jnp.zeros((STRIPE, D), f32)

    deg = _deg_call(ebt, ones_e, zvec)
    degT = deg.T

    hs1 = _scale_call(x_pad, degT)
    p1 = _agg_call(hs1, eb[0], eb[1], zblk)
    h1 = _mm1_call(p1, degT, W1, b1.reshape(1, D))
    p2 = _agg_call(h1, eb[0], eb[1], zblk)
    wt = jnp.pad(Wc[:D], ((0, 0), (0, DP - 2)))
    wb = jnp.pad(Wc[D:], ((0, 0), (0, DP - 2)))
    bcp = jnp.pad(bc.reshape(1, 2), ((0, 0), (0, DP - 2)))
    z, u, v = _mm2_call(p2, degT, W2, b2.reshape(1, D), wt, wb, bcp)

    o = _pair_call(u, v, p[0], p[1])
    logits = o.reshape(PP, DP)[:N_PAIRS, :2]
    return (z[:N_NODES], logits)


# speedup vs baseline: 2.9273x; 1.2435x over previous
import functools

import jax
import jax.numpy as jnp
from jax import lax
from jax.experimental import pallas as pl
from jax.experimental.pallas import tpu as pltpu
from jax.experimental.pallas import tpu_sc as plsc

N_NODES = 10000
N_EDGES = 320000
N_PAIRS = 100000
D = 128

NC = 2
NS = 16
NW = NC * NS
NP = 10240
STRIPE = NP // NS

EB = 128
EP = 327680
EBA = 128
AGG_W = NW
ROWS_PER_W = EP // EBA // AGG_W
CH = 16
NCH = ROWS_PER_W // CH
ROWS_PER_T = EP // EB // NS

PP = 102400
P_PER_W = PP // NW
P_ITER = P_PER_W // EB
DP = 16

BLK = 1024
GRID = NP // BLK

_mesh = plsc.VectorSubcoreMesh(core_axis_name="c", subcore_axis_name="s")


def _deg_body(eidx_h, ones_h, zvec_h, out_h, idx_v, ones_v, deg_sh):
    cid = lax.axis_index("c")
    sid = lax.axis_index("s")
    pltpu.sync_copy(ones_h, ones_v)
    pltpu.sync_copy(zvec_h, deg_sh.at[pl.ds(sid * STRIPE, STRIPE)])
    for m in range(2):
        pltpu.sync_copy(eidx_h.at[cid, 2 * sid + m],
                        idx_v.at[pl.ds(m * (ROWS_PER_T // 2), ROWS_PER_T // 2)])
    plsc.subcore_barrier()

    def step(j, _):
        pltpu.sync_copy(ones_v, deg_sh.at[idx_v.at[j]], add=True)
        return 0

    lax.fori_loop(0, ROWS_PER_T, step, 0)
    plsc.subcore_barrier()
    pltpu.sync_copy(deg_sh.at[pl.ds(sid * STRIPE, STRIPE)],
                    out_h.at[cid, pl.ds(sid * STRIPE, STRIPE)])


def _agg_body(tab_h, srcb_h, dstb_h, zblk_h, out_h,
              idx_s, idx_d, rows_v, rows1, acc_sh, sem, sem1):
    cid = lax.axis_index("c")
    sid = lax.axis_index("s")
    wid = cid * NS + sid
    pltpu.sync_copy(zblk_h, acc_sh.at[pl.ds(sid * STRIPE, STRIPE)])
    plsc.subcore_barrier()

    def chunk(c, _):
        pltpu.sync_copy(srcb_h.at[wid, pl.ds(c * CH, CH)], idx_s)
        pltpu.sync_copy(dstb_h.at[wid, pl.ds(c * CH, CH)], idx_d)
        pltpu.async_copy(tab_h.at[idx_s.at[0]], rows_v, sem)

        def step(jj, _):
            j = 2 * jj
            pltpu.async_copy(tab_h.at[idx_s.at[j + 1]], rows1, sem1)
            pltpu.make_async_copy(tab_h.at[idx_s.at[j]], rows_v, sem).wait()
            pltpu.sync_copy(rows_v, acc_sh.at[idx_d.at[j]], add=True)

            @pl.when(jj < CH // 2 - 1)
            def _():
                pltpu.async_copy(tab_h.at[idx_s.at[j + 2]], rows_v, sem)

            pltpu.make_async_copy(tab_h.at[idx_s.at[j + 1]], rows1, sem1).wait()
            pltpu.sync_copy(rows1, acc_sh.at[idx_d.at[j + 1]], add=True)
            return 0

        lax.fori_loop(0, CH // 2, step, 0)
        return 0

    lax.fori_loop(0, NCH, chunk, 0)
    plsc.subcore_barrier()
    pltpu.sync_copy(acc_sh.at[pl.ds(sid * STRIPE, STRIPE)],
                    out_h.at[cid, pl.ds(sid * STRIPE, STRIPE)])


def _pair_body(u_h, v_h, p0_h, p1_h, o_h,
               p0_v, p1_v, ur_v, vr_v, o_v, semu, semv):
    cid = lax.axis_index("c")
    sid = lax.axis_index("s")
    wid = cid * NS + sid
    pltpu.sync_copy(p0_h.at[wid], p0_v)
    pltpu.sync_copy(p1_h.at[wid], p1_v)

    def step(j, _):
        cu = pltpu.async_copy(u_h.at[p0_v.at[j]], ur_v, semu)
        cv = pltpu.async_copy(v_h.at[p1_v.at[j]], vr_v, semv)
        cu.wait()
        cv.wait()

        def add_row(k, _):
            o_v[k, :] = ur_v[k, :] + vr_v[k, :]
            return 0

        lax.fori_loop(0, EB, add_row, 0)
        pltpu.sync_copy(o_v, o_h.at[wid, pl.ds(j * EB, EB)])
        return 0

    lax.fori_loop(0, P_ITER, step, 0)


_deg_call = pl.kernel(
    _deg_body,
    out_type=jax.ShapeDtypeStruct((NC, NP), jnp.float32),
    mesh=_mesh,
    scratch_types=[
        pltpu.VMEM((ROWS_PER_T, EB), jnp.int32),
        pltpu.VMEM((EB,), jnp.float32),
        pltpu.VMEM_SHARED((NP,), jnp.float32),
    ],
)

_agg_call = pl.kernel(
    _agg_body,
    out_type=jax.ShapeDtypeStruct((NC, NP, D), jnp.float32),
    mesh=_mesh,
    scratch_types=[
        pltpu.VMEM((CH, EBA), jnp.int32),
        pltpu.VMEM((CH, EBA), jnp.int32),
        pltpu.VMEM((EBA, D), jnp.float32),
        pltpu.VMEM((EBA, D), jnp.float32),
        pltpu.VMEM_SHARED((NP, D), jnp.float32),
        pltpu.SemaphoreType.DMA,
        pltpu.SemaphoreType.DMA,
    ],
)

_pair_call = pl.kernel(
    _pair_body,
    out_type=jax.ShapeDtypeStruct((NW, P_PER_W, DP), jnp.float32),
    mesh=_mesh,
    scratch_types=[
        pltpu.VMEM((P_ITER, EB), jnp.int32),
        pltpu.VMEM((P_ITER, EB), jnp.int32),
        pltpu.VMEM((EB, DP), jnp.float32),
        pltpu.VMEM((EB, DP), jnp.float32),
        pltpu.VMEM((EB, DP), jnp.float32),
        pltpu.SemaphoreType.DMA,
        pltpu.SemaphoreType.DMA,
    ],
    compiler_params=pltpu.CompilerParams(use_tc_tiling_on_sc=False),
)


def _scale_body(x_ref, dT_ref, o_ref):
    a = lax.rsqrt(jnp.maximum(dT_ref[:, 0:1], 1.0))
    o_ref[...] = x_ref[...] * a


def _mm1_body(p_ref, dT_ref, W_ref, b_ref, o_ref):
    acc = p_ref[0] + p_ref[1]
    dT = dT_ref[...]
    bsc = lax.rsqrt(jnp.maximum(dT[:, 1:2], 1.0))
    asc = lax.rsqrt(jnp.maximum(dT[:, 0:1], 1.0))
    y = jnp.dot(acc * bsc, W_ref[...], preferred_element_type=jnp.float32)
    o_ref[...] = jnp.maximum(y + b_ref[...], 0.0) * asc


def _mm2_body(p_ref, dT_ref, W_ref, b_ref, wt_ref, wb_ref, bc_ref,
              z_ref, u_ref, v_ref):
    acc = p_ref[0] + p_ref[1]
    dT = dT_ref[...]
    bsc = lax.rsqrt(jnp.maximum(dT[:, 1:2], 1.0))
    y = jnp.dot(acc * bsc, W_ref[...], preferred_element_type=jnp.float32)
    z = jnp.maximum(y + b_ref[...], 0.0)
    z_ref[...] = z
    u_ref[...] = jnp.dot(z, wt_ref[...], preferred_element_type=jnp.float32) + bc_ref[...]
    v_ref[...] = jnp.dot(z, wb_ref[...], preferred_element_type=jnp.float32)


def _row_spec():
    return pl.BlockSpec((BLK, D), lambda i: (i, 0))


def _deg_spec():
    return pl.BlockSpec((BLK, 2), lambda i: (i, 0))


def _full(shape):
    return pl.BlockSpec(shape, lambda i: tuple(0 for _ in shape))


_scale_call = pl.pallas_call(
    _scale_body,
    grid=(GRID,),
    in_specs=[_row_spec(), _deg_spec()],
    out_specs=_row_spec(),
    out_shape=jax.ShapeDtypeStruct((NP, D), jnp.float32),
)

_mm1_call = pl.pallas_call(
    _mm1_body,
    grid=(GRID,),
    in_specs=[pl.BlockSpec((2, BLK, D), lambda i: (0, i, 0)), _deg_spec(),
              _full((D, D)), _full((1, D))],
    out_specs=_row_spec(),
    out_shape=jax.ShapeDtypeStruct((NP, D), jnp.float32),
)

_mm2_call = pl.pallas_call(
    _mm2_body,
    grid=(GRID,),
    in_specs=[pl.BlockSpec((2, BLK, D), lambda i: (0, i, 0)), _deg_spec(),
              _full((D, D)), _full((1, D)),
              _full((D, DP)), _full((D, DP)), _full((1, DP))],
    out_specs=[_row_spec(),
               pl.BlockSpec((BLK, DP), lambda i: (i, 0)),
               pl.BlockSpec((BLK, DP), lambda i: (i, 0))],
    out_shape=[jax.ShapeDtypeStruct((NP, D), jnp.float32),
               jax.ShapeDtypeStruct((NP, DP), jnp.float32),
               jax.ShapeDtypeStruct((NP, DP), jnp.float32)],
)


def kernel(x, edge_index, pair_index, W1, b1, W2, b2, Wc, bc):
    f32 = jnp.float32
    x_pad = jnp.pad(x, ((0, NP - N_NODES), (0, 0)))

    pad_ids = N_NODES + jnp.arange(EP - N_EDGES, dtype=jnp.int32) % (NP - N_NODES)
    e = jnp.concatenate([edge_index, jnp.stack([pad_ids, pad_ids])], axis=1)
    eb = e.reshape(2, AGG_W, ROWS_PER_W, EBA)
    ebt = e.reshape(2, NS * 2, ROWS_PER_T // 2, EB)

    pad_pids = jnp.arange(PP - N_PAIRS, dtype=jnp.int32) % N_NODES
    p = jnp.concatenate([pair_index, jnp.stack([pad_pids, pad_pids])],
                        axis=1).reshape(2, NW, P_ITER, EB)

    ones_e = jnp.ones((EB,), f32)
    zvec = jnp.zeros((STRIPE,), f32)
    zblk = jnp.zeros((STRIPE, D), f32)

    deg = _deg_call(ebt, ones_e, zvec)
    degT = deg.T

    hs1 = _scale_call(x_pad, degT)
    p1 = _agg_call(hs1, eb[0], eb[1], zblk)
    h1 = _mm1_call(p1, degT, W1, b1.reshape(1, D))
    p2 = _agg_call(h1, eb[0], eb[1], zblk)
    wt = jnp.pad(Wc[:D], ((0, 0), (0, DP - 2)))
    wb = jnp.pad(Wc[D:], ((0, 0), (0, DP - 2)))
    bcp = jnp.pad(bc.reshape(1, 2), ((0, 0), (0, DP - 2)))
    z, u, v = _mm2_call(p2, degT, W2, b2.reshape(1, D), wt, wb, bcp)

    o = _pair_call(u, v, p[0], p[1])
    logits = o.reshape(PP, DP)[:N_PAIRS, :2]
    return (z[:N_NODES], logits)


# speedup vs baseline: 3.2286x; 1.1029x over previous
import functools

import jax
import jax.numpy as jnp
from jax import lax
from jax.experimental import pallas as pl
from jax.experimental.pallas import tpu as pltpu
from jax.experimental.pallas import tpu_sc as plsc

N_NODES = 10000
N_EDGES = 320000
N_PAIRS = 100000
D = 128

NC = 2
NS = 16
NW = NC * NS
NP = 10240
STRIPE = NP // NS

EB = 128
EP = 327680
EBA = 128
AGG_W = NW
ROWS_PER_W = EP // EBA // AGG_W
CH = 40
NCH = ROWS_PER_W // CH
ROWS_PER_T = EP // EB // NS

PP = 106496
P_PER_W = PP // NW
P_ITER = P_PER_W // EB
DP = 16

BLK = 1024
GRID = NP // BLK

_mesh = plsc.VectorSubcoreMesh(core_axis_name="c", subcore_axis_name="s")


def _deg_body(eidx_h, ones_h, zvec_h, out_h, idx_v, ones_v, deg_sh, semd):
    cid = lax.axis_index("c")
    sid = lax.axis_index("s")
    pltpu.sync_copy(ones_h, ones_v)
    pltpu.sync_copy(zvec_h, deg_sh.at[pl.ds(sid * STRIPE, STRIPE)])
    for m in range(2):
        pltpu.sync_copy(eidx_h.at[cid, 2 * sid + m],
                        idx_v.at[pl.ds(m * (ROWS_PER_T // 2), ROWS_PER_T // 2)])
    plsc.subcore_barrier()

    GRP = 16

    def group(g, _):
        def fire(i, _):
            pltpu.async_copy(ones_v, deg_sh.at[idx_v.at[g * GRP + i]], semd,
                             add=True)
            return 0

        lax.fori_loop(0, GRP, fire, 0)

        def drain(i, _):
            pltpu.make_async_copy(ones_v, deg_sh.at[idx_v.at[g * GRP + i]],
                                  semd).wait()
            return 0

        lax.fori_loop(0, GRP, drain, 0)
        return 0

    lax.fori_loop(0, ROWS_PER_T // GRP, group, 0)
    plsc.subcore_barrier()
    pltpu.sync_copy(deg_sh.at[pl.ds(sid * STRIPE, STRIPE)],
                    out_h.at[cid, pl.ds(sid * STRIPE, STRIPE)])


def _agg_body(tab_h, srcb_h, dstb_h, zblk_h, out_h,
              idx_s, idx_d, rows_v, rows1, acc_sh, sem, sem1):
    cid = lax.axis_index("c")
    sid = lax.axis_index("s")
    wid = cid * NS + sid
    pltpu.sync_copy(zblk_h, acc_sh.at[pl.ds(sid * STRIPE, STRIPE)])
    plsc.subcore_barrier()

    def chunk(c, _):
        pltpu.sync_copy(srcb_h.at[wid, pl.ds(c * CH, CH)], idx_s)
        pltpu.sync_copy(dstb_h.at[wid, pl.ds(c * CH, CH)], idx_d)
        pltpu.async_copy(tab_h.at[idx_s.at[0]], rows_v, sem)

        def step(jj, _):
            j = 2 * jj
            pltpu.async_copy(tab_h.at[idx_s.at[j + 1]], rows1, sem1)
            pltpu.make_async_copy(tab_h.at[idx_s.at[j]], rows_v, sem).wait()
            pltpu.sync_copy(rows_v, acc_sh.at[idx_d.at[j]], add=True)

            @pl.when(jj < CH // 2 - 1)
            def _():
                pltpu.async_copy(tab_h.at[idx_s.at[j + 2]], rows_v, sem)

            pltpu.make_async_copy(tab_h.at[idx_s.at[j + 1]], rows1, sem1).wait()
            pltpu.sync_copy(rows1, acc_sh.at[idx_d.at[j + 1]], add=True)
            return 0

        lax.fori_loop(0, CH // 2, step, 0)
        return 0

    lax.fori_loop(0, NCH, chunk, 0)
    plsc.subcore_barrier()
    pltpu.sync_copy(acc_sh.at[pl.ds(sid * STRIPE, STRIPE)],
                    out_h.at[cid, pl.ds(sid * STRIPE, STRIPE)])


def _pair_body(u_h, v_h, p0_h, p1_h, o_h,
               p0_v, p1_v, ur0, vr0, o0_v, ur1, vr1, o1_v,
               su0, sv0, su1, sv1):
    cid = lax.axis_index("c")
    sid = lax.axis_index("s")
    wid = cid * NS + sid
    pltpu.sync_copy(p0_h.at[wid], p0_v)
    pltpu.sync_copy(p1_h.at[wid], p1_v)

    def halfstep(j, ur, vr, ov, su, sv, nxt):
        pltpu.make_async_copy(u_h.at[p0_v.at[j]], ur, su).wait()
        pltpu.make_async_copy(v_h.at[p1_v.at[j]], vr, sv).wait()

        def add_row(k, _):
            ov[k, :] = ur[k, :] + vr[k, :]
            return 0

        lax.fori_loop(0, EB, add_row, 0)

        @pl.when(nxt < P_ITER)
        def _():
            pltpu.async_copy(u_h.at[p0_v.at[nxt]], ur, su)
            pltpu.async_copy(v_h.at[p1_v.at[nxt]], vr, sv)

        pltpu.sync_copy(ov, o_h.at[wid, pl.ds(j * EB, EB)])

    pltpu.async_copy(u_h.at[p0_v.at[0]], ur0, su0)
    pltpu.async_copy(v_h.at[p1_v.at[0]], vr0, sv0)
    pltpu.async_copy(u_h.at[p0_v.at[1]], ur1, su1)
    pltpu.async_copy(v_h.at[p1_v.at[1]], vr1, sv1)

    def step(jj, _):
        j = 2 * jj
        halfstep(j, ur0, vr0, o0_v, su0, sv0, j + 2)
        halfstep(j + 1, ur1, vr1, o1_v, su1, sv1, j + 3)
        return 0

    lax.fori_loop(0, P_ITER // 2, step, 0)


_deg_call = pl.kernel(
    _deg_body,
    out_type=jax.ShapeDtypeStruct((NC, NP), jnp.float32),
    mesh=_mesh,
    scratch_types=[
        pltpu.VMEM((ROWS_PER_T, EB), jnp.int32),
        pltpu.VMEM((EB,), jnp.float32),
        pltpu.VMEM_SHARED((NP,), jnp.float32),
        pltpu.SemaphoreType.DMA,
    ],
)

_agg_call = pl.kernel(
    _agg_body,
    out_type=jax.ShapeDtypeStruct((NC, NP, D), jnp.float32),
    mesh=_mesh,
    scratch_types=[
        pltpu.VMEM((CH, EBA), jnp.int32),
        pltpu.VMEM((CH, EBA), jnp.int32),
        pltpu.VMEM((EBA, D), jnp.float32),
        pltpu.VMEM((EBA, D), jnp.float32),
        pltpu.VMEM_SHARED((NP, D), jnp.float32),
        pltpu.SemaphoreType.DMA,
        pltpu.SemaphoreType.DMA,
    ],
)

_pair_call = pl.kernel(
    _pair_body,
    out_type=jax.ShapeDtypeStruct((NW, P_PER_W, DP), jnp.float32),
    mesh=_mesh,
    scratch_types=[
        pltpu.VMEM((P_ITER, EB), jnp.int32),
        pltpu.VMEM((P_ITER, EB), jnp.int32),
        pltpu.VMEM((EB, DP), jnp.float32),
        pltpu.VMEM((EB, DP), jnp.float32),
        pltpu.VMEM((EB, DP), jnp.float32),
        pltpu.VMEM((EB, DP), jnp.float32),
        pltpu.VMEM((EB, DP), jnp.float32),
        pltpu.VMEM((EB, DP), jnp.float32),
        pltpu.SemaphoreType.DMA,
        pltpu.SemaphoreType.DMA,
        pltpu.SemaphoreType.DMA,
        pltpu.SemaphoreType.DMA,
    ],
    compiler_params=pltpu.CompilerParams(use_tc_tiling_on_sc=False),
)


def _scale_body(x_ref, dT_ref, o_ref):
    a = lax.rsqrt(jnp.maximum(dT_ref[:, 0:1], 1.0))
    o_ref[...] = x_ref[...] * a


def _mm1_body(p_ref, dT_ref, W_ref, b_ref, o_ref):
    acc = p_ref[0] + p_ref[1]
    dT = dT_ref[...]
    bsc = lax.rsqrt(jnp.maximum(dT[:, 1:2], 1.0))
    asc = lax.rsqrt(jnp.maximum(dT[:, 0:1], 1.0))
    y = jnp.dot(acc * bsc, W_ref[...], preferred_element_type=jnp.float32)
    o_ref[...] = jnp.maximum(y + b_ref[...], 0.0) * asc


def _mm2_body(p_ref, dT_ref, W_ref, b_ref, wt_ref, wb_ref, bc_ref,
              z_ref, u_ref, v_ref):
    acc = p_ref[0] + p_ref[1]
    dT = dT_ref[...]
    bsc = lax.rsqrt(jnp.maximum(dT[:, 1:2], 1.0))
    y = jnp.dot(acc * bsc, W_ref[...], preferred_element_type=jnp.float32)
    z = jnp.maximum(y + b_ref[...], 0.0)
    z_ref[...] = z
    u_ref[...] = jnp.dot(z, wt_ref[...], preferred_element_type=jnp.float32) + bc_ref[...]
    v_ref[...] = jnp.dot(z, wb_ref[...], preferred_element_type=jnp.float32)


def _row_spec():
    return pl.BlockSpec((BLK, D), lambda i: (i, 0))


def _deg_spec():
    return pl.BlockSpec((BLK, 2), lambda i: (i, 0))


def _full(shape):
    return pl.BlockSpec(shape, lambda i: tuple(0 for _ in shape))


_scale_call = pl.pallas_call(
    _scale_body,
    grid=(GRID,),
    in_specs=[_row_spec(), _deg_spec()],
    out_specs=_row_spec(),
    out_shape=jax.ShapeDtypeStruct((NP, D), jnp.float32),
)

_mm1_call = pl.pallas_call(
    _mm1_body,
    grid=(GRID,),
    in_specs=[pl.BlockSpec((2, BLK, D), lambda i: (0, i, 0)), _deg_spec(),
              _full((D, D)), _full((1, D))],
    out_specs=_row_spec(),
    out_shape=jax.ShapeDtypeStruct((NP, D), jnp.float32),
)

_mm2_call = pl.pallas_call(
    _mm2_body,
    grid=(GRID,),
    in_specs=[pl.BlockSpec((2, BLK, D), lambda i: (0, i, 0)), _deg_spec(),
              _full((D, D)), _full((1, D)),
              _full((D, DP)), _full((D, DP)), _full((1, DP))],
    out_specs=[_row_spec(),
               pl.BlockSpec((BLK, DP), lambda i: (i, 0)),
               pl.BlockSpec((BLK, DP), lambda i: (i, 0))],
    out_shape=[jax.ShapeDtypeStruct((N_NODES, D), jnp.float32),
               jax.ShapeDtypeStruct((NP, DP), jnp.float32),
               jax.ShapeDtypeStruct((NP, DP), jnp.float32)],
)


def kernel(x, edge_index, pair_index, W1, b1, W2, b2, Wc, bc):
    f32 = jnp.float32
    x_pad = jnp.pad(x, ((0, NP - N_NODES), (0, 0)))

    pad_ids = N_NODES + jnp.arange(EP - N_EDGES, dtype=jnp.int32) % (NP - N_NODES)
    e = jnp.concatenate([edge_index, jnp.stack([pad_ids, pad_ids])], axis=1)
    eb = e.reshape(2, AGG_W, ROWS_PER_W, EBA)
    ebt = e.reshape(2, NS * 2, ROWS_PER_T // 2, EB)

    pad_pids = jnp.arange(PP - N_PAIRS, dtype=jnp.int32) % N_NODES
    p = jnp.concatenate([pair_index, jnp.stack([pad_pids, pad_pids])],
                        axis=1).reshape(2, NW, P_ITER, EB)

    ones_e = jnp.ones((EB,), f32)
    zvec = jnp.zeros((STRIPE,), f32)
    zblk = jnp.zeros((STRIPE, D), f32)

    deg = _deg_call(ebt, ones_e, zvec)
    degT = deg.T

    hs1 = _scale_call(x_pad, degT)
    p1 = _agg_call(hs1, eb[0], eb[1], zblk)
    h1 = _mm1_call(p1, degT, W1, b1.reshape(1, D))
    p2 = _agg_call(h1, eb[0], eb[1], zblk)
    wt = jnp.pad(Wc[:D], ((0, 0), (0, DP - 2)))
    wb = jnp.pad(Wc[D:], ((0, 0), (0, DP - 2)))
    bcp = jnp.pad(bc.reshape(1, 2), ((0, 0), (0, DP - 2)))
    z, u, v = _mm2_call(p2, degT, W2, b2.reshape(1, D), wt, wb, bcp)

    o = _pair_call(u, v, p[0], p[1])
    logits = o.reshape(PP, DP)[:N_PAIRS, :2]
    return (z, logits)
